# baseline (speedup 1.0000x reference)
import functools

import jax
import jax.numpy as jnp
import numpy as np
from jax import lax
from jax.experimental import pallas as pl
from jax.experimental.pallas import tpu as pltpu
from jax.experimental.pallas import tpu_sc as plsc

_PS = 4
_GRID = 32
_AIR = (102, 576, 3352)
_NTOK = 3717
_HIST = 3728

_NC, _NS = 2, 16
_NW = _NC * _NS
_L = 64

_LOGTAB = np.zeros(80, np.float32)
_LOGTAB[1:65] = np.log(np.arange(1, 65, dtype=np.float64)).astype(np.float32)


def _sc_body(flat_hbm, logtab_hbm, out_hbm, data_v, hist_v, logtab_v, out_v):
    pw = data_v.shape[0]
    npatch = pw // _L
    wid = lax.axis_index("c") * _NS + lax.axis_index("s")

    pltpu.sync_copy(flat_hbm.at[pl.ds(wid * pw, pw)], data_v)
    pltpu.sync_copy(logtab_hbm, logtab_v)

    zeros16 = jnp.zeros((16,), jnp.int32)
    ones16 = jnp.ones((16,), jnp.int32)
    air_idx = [jnp.full((16,), a, jnp.int32) for a in _AIR]

    def zero_body(j, carry):
        hist_v[pl.ds(j * 16, 16)] = zeros16
        return carry
    lax.fori_loop(0, _HIST // 16, zero_body, 0)

    def patch_body(i, acc):
        base = i * _L
        xs = [data_v[pl.ds(base + 16 * k, 16)] for k in range(4)]
        for x in xs:
            plsc.addupdate_scatter(hist_v, [x], ones16)
        n_air = plsc.load_gather(hist_v, [air_idx[0]])
        n_air = n_air + plsc.load_gather(hist_v, [air_idx[1]])
        n_air = n_air + plsc.load_gather(hist_v, [air_idx[2]])
        s_vec = jnp.full((16,), _L, jnp.int32) - n_air
        cs = [plsc.load_gather(hist_v, [x]) for x in xs]
        for x in xs:
            plsc.store_scatter(hist_v, [x], zeros16)
        log_s = plsc.load_gather(logtab_v, [s_vec])
        s_f = s_vec.astype(jnp.float32)
        recip = 1.0 / jnp.maximum(s_f, 1.0)
        inner = jnp.zeros((16,), jnp.float32)
        for k in range(4):
            nonair = ((xs[k] != _AIR[0]) & (xs[k] != _AIR[1])
                      & (xs[k] != _AIR[2]))
            log_c = plsc.load_gather(logtab_v, [cs[k]])
            inner = inner + jnp.where(nonair, log_s - log_c, 0.0)
        return acc + inner * recip

    acc = lax.fori_loop(0, npatch, patch_body, jnp.zeros((16,), jnp.float32))
    out_v[...] = acc
    pltpu.sync_copy(out_v, out_hbm.at[wid])


@jax.jit
def _sc_entropy(flat, logtab):
    pw = flat.shape[0] // _NW
    fn = functools.partial(
        pl.kernel,
        out_type=jax.ShapeDtypeStruct((_NW, 16), jnp.float32),
        mesh=plsc.VectorSubcoreMesh(
            core_axis_name="c", subcore_axis_name="s",
            num_cores=_NC, num_subcores=_NS),
        scratch_types=[
            pltpu.VMEM((pw,), jnp.int32),
            pltpu.VMEM((_HIST,), jnp.int32),
            pltpu.VMEM((80,), jnp.float32),
            pltpu.VMEM((16,), jnp.float32),
        ],
        compiler_params=pltpu.CompilerParams(needs_layout_passes=False),
    )(_sc_body)
    return fn(flat, logtab)


def kernel(structure):
    B = structure.shape[0]
    n = _GRID // _PS
    num_patches = n * n * n
    p = structure.reshape(B, n, _PS, n, _PS, n, _PS)
    p = jnp.transpose(p, (0, 1, 3, 5, 2, 4, 6)).reshape(-1)
    partials = _sc_entropy(p, jnp.asarray(_LOGTAB))
    total = jnp.sum(partials)
    return total / (B * num_patches + 1e-06)


# speedup vs baseline: 16.5112x; 16.5112x over previous
import functools

import jax
import jax.numpy as jnp
import numpy as np
from jax import lax
from jax.experimental import pallas as pl
from jax.experimental.pallas import tpu as pltpu
from jax.experimental.pallas import tpu_sc as plsc

_PS = 4
_GRID = 32
_AIR = (102, 576, 3352)
_HREG = 3728
_NHIST = 8

_NC, _NS = 2, 16
_NW = _NC * _NS
_L = 64

_LOGTAB = np.zeros(80, np.float32)
_LOGTAB[1:65] = np.log(np.arange(1, 65, dtype=np.float64)).astype(np.float32)


def _sc_body(flat_hbm, logtab_hbm, out_hbm, data_v, hist_v, logtab_v, out_v):
    pw = data_v.shape[0]
    wid = lax.axis_index("c") * _NS + lax.axis_index("s")

    pltpu.sync_copy(flat_hbm.at[pl.ds(wid * pw, pw)], data_v)
    pltpu.sync_copy(logtab_hbm, logtab_v)

    zeros16 = jnp.zeros((16,), jnp.int32)
    zeros16f = jnp.zeros((16,), jnp.float32)
    ones16 = jnp.ones((16,), jnp.int32)
    full64 = jnp.full((16,), _L, jnp.int32)

    def zero_body(j, carry):
        hist_v[pl.ds(j * 16, 16)] = zeros16
        return carry
    lax.fori_loop(0, _NHIST * _HREG // 16, zero_body, 0)

    lane = lax.iota(jnp.int32, 16)
    pat = [(lane >> 2) * _HREG, (lane >> 2) * _HREG + 4 * _HREG]
    airp = [[p + a for a in _AIR] for p in pat]

    def group_body(g, acc):
        base = ((g >> 6) * 32768 + ((g >> 3) & 7) * 4096 + (g & 7) * 128)
        rows = [base + a * 1024 + c * 32 for a in range(_PS)
                for c in range(_PS)]
        for r in rows:
            for h in (0, 1):
                x = data_v[pl.ds(r + 16 * h, 16)]
                plsc.addupdate_scatter(hist_v, [x + pat[h]], ones16)
        logs, recip = [], []
        for h in (0, 1):
            n_air = plsc.load_gather(hist_v, [airp[h][0]])
            n_air = n_air + plsc.load_gather(hist_v, [airp[h][1]])
            n_air = n_air + plsc.load_gather(hist_v, [airp[h][2]])
            s_vec = full64 - n_air
            logs.append(plsc.load_gather(logtab_v, [s_vec]))
            recip.append(1.0 / jnp.maximum(s_vec.astype(jnp.float32), 1.0))
        for h in (0, 1):
            inner = zeros16f
            for r in rows:
                x = data_v[pl.ds(r + 16 * h, 16)]
                cv = plsc.load_gather(hist_v, [x + pat[h]])
                log_c = plsc.load_gather(logtab_v, [cv])
                nonair = ((x != _AIR[0]) & (x != _AIR[1]) & (x != _AIR[2]))
                inner = inner + jnp.where(nonair, logs[h] - log_c, 0.0)
            acc = acc + inner * recip[h]
        for r in rows:
            for h in (0, 1):
                x = data_v[pl.ds(r + 16 * h, 16)]
                plsc.store_scatter(hist_v, [x + pat[h]], zeros16)
        return acc

    n_groups = pw // (16 * 32)
    acc = lax.fori_loop(0, n_groups, group_body, zeros16f)
    out_v[...] = acc
    pltpu.sync_copy(out_v, out_hbm.at[wid])


@jax.jit
def _sc_entropy(flat, logtab):
    pw = flat.shape[0] // _NW
    fn = functools.partial(
        pl.kernel,
        out_type=jax.ShapeDtypeStruct((_NW, 16), jnp.float32),
        mesh=plsc.VectorSubcoreMesh(
            core_axis_name="c", subcore_axis_name="s",
            num_cores=_NC, num_subcores=_NS),
        scratch_types=[
            pltpu.VMEM((pw,), jnp.int32),
            pltpu.VMEM((_NHIST * _HREG,), jnp.int32),
            pltpu.VMEM((80,), jnp.float32),
            pltpu.VMEM((16,), jnp.float32),
        ],
        compiler_params=pltpu.CompilerParams(needs_layout_passes=False),
    )(_sc_body)
    return fn(flat, logtab)


def kernel(structure):
    B = structure.shape[0]
    n = _GRID // _PS
    num_patches = n * n * n
    partials = _sc_entropy(structure.reshape(-1), jnp.asarray(_LOGTAB))
    total = jnp.sum(partials)
    return total / (B * num_patches + 1e-06)


# speedup vs baseline: 26.0492x; 1.5777x over previous
import functools

import jax
import jax.numpy as jnp
import numpy as np
from jax import lax
from jax.experimental import pallas as pl
from jax.experimental.pallas import tpu as pltpu
from jax.experimental.pallas import tpu_sc as plsc

_PS = 4
_GRID = 32
_AIR = (102, 576, 3352)
_HREG = 3728
_NHIST = 8

_NC, _NS = 2, 16
_NW = _NC * _NS
_L = 64

_LOGTAB = np.zeros(80, np.float32)
_LOGTAB[1:65] = np.log(np.arange(1, 65, dtype=np.float64)).astype(np.float32)


def _sc_body(flat_hbm, logtab_hbm, out_hbm, data_v, hist_v, logtab_v, out_v):
    pw = data_v.shape[0]
    wid = lax.axis_index("c") * _NS + lax.axis_index("s")

    pltpu.sync_copy(flat_hbm.at[pl.ds(wid * pw, pw)], data_v)
    pltpu.sync_copy(logtab_hbm, logtab_v)

    zeros16 = jnp.zeros((16,), jnp.int32)
    zeros16f = jnp.zeros((16,), jnp.float32)
    ones16 = jnp.ones((16,), jnp.int32)
    full64 = jnp.full((16,), _L, jnp.int32)

    def zero_body(j, carry):
        hist_v[pl.ds(j * 16, 16)] = zeros16
        return carry
    lax.fori_loop(0, _NHIST * _HREG // 16, zero_body, 0)

    lane = lax.iota(jnp.int32, 16)
    pat = [(lane >> 2) * _HREG, (lane >> 2) * _HREG + 4 * _HREG]
    airp = [[p + a for a in _AIR] for p in pat]

    def group_body(g, acc):
        base = ((g >> 6) * 32768 + ((g >> 3) & 7) * 4096 + (g & 7) * 128)
        rows = [base + a * 1024 + c * 32 for a in range(_PS)
                for c in range(_PS)]
        idxs = []
        for r in rows:
            for h in (0, 1):
                x = data_v[pl.ds(r + 16 * h, 16)]
                idxs.append((x + pat[h], h))
        for idx, _ in idxs:
            plsc.addupdate_scatter(hist_v, [idx], ones16)
        logs, recip = [], []
        for h in (0, 1):
            n_air = plsc.load_gather(hist_v, [airp[h][0]])
            n_air = n_air + plsc.load_gather(hist_v, [airp[h][1]])
            n_air = n_air + plsc.load_gather(hist_v, [airp[h][2]])
            s_vec = full64 - n_air
            logs.append(plsc.load_gather(logtab_v, [s_vec]))
            recip.append(1.0 / jnp.maximum(s_vec.astype(jnp.float32), 1.0))
        inner = [zeros16f, zeros16f]
        for idx, h in idxs:
            cv = plsc.load_gather(hist_v, [idx])
            log_c = plsc.load_gather(logtab_v, [cv])
            nonair = ((idx != airp[h][0]) & (idx != airp[h][1])
                      & (idx != airp[h][2]))
            inner[h] = inner[h] + jnp.where(nonair, logs[h] - log_c, 0.0)
        acc = acc + inner[0] * recip[0] + inner[1] * recip[1]
        for idx, _ in idxs:
            plsc.store_scatter(hist_v, [idx], zeros16)
        return acc

    n_groups = pw // (16 * 32)
    acc = lax.fori_loop(0, n_groups, group_body, zeros16f)
    out_v[...] = acc
    pltpu.sync_copy(out_v, out_hbm.at[wid])


@jax.jit
def _sc_entropy(flat, logtab):
    pw = flat.shape[0] // _NW
    fn = functools.partial(
        pl.kernel,
        out_type=jax.ShapeDtypeStruct((_NW, 16), jnp.float32),
        mesh=plsc.VectorSubcoreMesh(
            core_axis_name="c", subcore_axis_name="s",
            num_cores=_NC, num_subcores=_NS),
        scratch_types=[
            pltpu.VMEM((pw,), jnp.int32),
            pltpu.VMEM((_NHIST * _HREG,), jnp.int32),
            pltpu.VMEM((80,), jnp.float32),
            pltpu.VMEM((16,), jnp.float32),
        ],
        compiler_params=pltpu.CompilerParams(needs_layout_passes=False),
    )(_sc_body)
    return fn(flat, logtab)


def kernel(structure):
    B = structure.shape[0]
    n = _GRID // _PS
    num_patches = n * n * n
    partials = _sc_entropy(structure.reshape(-1), jnp.asarray(_LOGTAB))
    total = jnp.sum(partials)
    return total / (B * num_patches + 1e-06)
